# baseline (speedup 1.0000x reference)
import jax
import jax.numpy as jnp
from jax import lax
from jax.experimental import pallas as pl
from jax.experimental.pallas import tpu as pltpu
from jax.experimental.pallas import tpu_sc as plsc

MAX_LEN = 4096
D_MODEL = 1024
BATCH = 4

NUM_CORES = 2
NUM_SUBCORES = 16
NUM_WORKERS = NUM_CORES * NUM_SUBCORES
ROWS_PER_WORKER = MAX_LEN // NUM_WORKERS
CHUNK = 32
NUM_CHUNKS = ROWS_PER_WORKER // CHUNK


def _sc_broadcast(table_hbm, out_hbm, buf0, buf1, buf2, gsem, ssem):
    wid = lax.axis_index("s") * NUM_CORES + lax.axis_index("c")
    base = wid * ROWS_PER_WORKER
    bufs = (buf0, buf1, buf2)

    def gather(c):
        row = base + c * CHUNK
        return pltpu.async_copy(table_hbm.at[pl.ds(row, CHUNK)], bufs[c % 3], gsem)

    gathers = {0: gather(0)}
    scatters = {}
    for c in range(NUM_CHUNKS):
        row = base + c * CHUNK
        gathers[c].wait()
        scatters[c] = [
            pltpu.async_copy(bufs[c % 3], out_hbm.at[b, pl.ds(row, CHUNK)], ssem)
            for b in range(BATCH)
        ]
        if c + 1 < NUM_CHUNKS:
            if c - 2 >= 0:
                for cp in scatters[c - 2]:
                    cp.wait()
            gathers[c + 1] = gather(c + 1)
    for c in range(max(0, NUM_CHUNKS - 2), NUM_CHUNKS):
        for cp in scatters[c]:
            cp.wait()


def _sc_broadcast_direct(table_hbm, out_hbm, sem):
    wid = lax.axis_index("s") * NUM_CORES + lax.axis_index("c")
    base = wid * ROWS_PER_WORKER
    sl = pl.ds(base, ROWS_PER_WORKER)
    copies = [
        pltpu.async_copy(table_hbm.at[sl], out_hbm.at[b, sl], sem)
        for b in range(BATCH)
    ]
    for cp in copies:
        cp.wait()


def kernel(x, emb_weight):
    del x
    f = pl.kernel(
        _sc_broadcast_direct,
        out_type=jax.ShapeDtypeStruct((BATCH, MAX_LEN, D_MODEL), jnp.float32),
        mesh=plsc.VectorSubcoreMesh(core_axis_name="c", subcore_axis_name="s"),
        scratch_types=[
            pltpu.SemaphoreType.DMA,
        ],
    )
    return f(emb_weight)


# speedup vs baseline: 37.9603x; 37.9603x over previous
import jax
import jax.numpy as jnp
from jax import lax
from jax.experimental import pallas as pl
from jax.experimental.pallas import tpu as pltpu
from jax.experimental.pallas import tpu_sc as plsc

MAX_LEN = 4096
D_MODEL = 1024
BATCH = 4

NUM_CORES = 2
NUM_SUBCORES = 16
NUM_WORKERS = NUM_CORES * NUM_SUBCORES
ROWS_PER_WORKER = MAX_LEN // NUM_WORKERS
CHUNK = 32
NUM_CHUNKS = ROWS_PER_WORKER // CHUNK


def _sc_broadcast(table_hbm, out_hbm, buf0, buf1, buf2, gsem, ssem):
    wid = lax.axis_index("s") * NUM_CORES + lax.axis_index("c")
    base = wid * ROWS_PER_WORKER
    bufs = (buf0, buf1, buf2)

    def gather(c):
        row = base + c * CHUNK
        return pltpu.async_copy(table_hbm.at[pl.ds(row, CHUNK)], bufs[c % 3], gsem)

    gathers = {0: gather(0)}
    scatters = {}
    for c in range(NUM_CHUNKS):
        row = base + c * CHUNK
        gathers[c].wait()
        scatters[c] = [
            pltpu.async_copy(bufs[c % 3], out_hbm.at[b, pl.ds(row, CHUNK)], ssem)
            for b in range(BATCH)
        ]
        if c + 1 < NUM_CHUNKS:
            if c - 2 >= 0:
                for cp in scatters[c - 2]:
                    cp.wait()
            gathers[c + 1] = gather(c + 1)
    for c in range(max(0, NUM_CHUNKS - 2), NUM_CHUNKS):
        for cp in scatters[c]:
            cp.wait()


def _sc_broadcast_direct(table_hbm, out_hbm, sem):
    wid = lax.axis_index("s") * NUM_CORES + lax.axis_index("c")
    base = wid * ROWS_PER_WORKER
    sl = pl.ds(base, ROWS_PER_WORKER)
    copies = [
        pltpu.async_copy(table_hbm.at[sl], out_hbm.at[b, sl], sem)
        for b in range(BATCH)
    ]
    for cp in copies:
        cp.wait()


ROW_BLOCK = 256


def _tc_body(in_ref, out_ref):
    out_ref[...] = in_ref[...][None]


def kernel(x, emb_weight):
    del x
    grid = (MAX_LEN // ROW_BLOCK, BATCH)
    return pl.pallas_call(
        _tc_body,
        grid=grid,
        in_specs=[pl.BlockSpec((ROW_BLOCK, D_MODEL), lambda i, b: (i, 0))],
        out_specs=pl.BlockSpec((1, ROW_BLOCK, D_MODEL), lambda i, b: (b, i, 0)),
        out_shape=jax.ShapeDtypeStruct((BATCH, MAX_LEN, D_MODEL), jnp.float32),
    )(emb_weight)


# speedup vs baseline: 53.6156x; 1.4124x over previous
import jax
import jax.numpy as jnp
from jax import lax
from jax.experimental import pallas as pl
from jax.experimental.pallas import tpu as pltpu
from jax.experimental.pallas import tpu_sc as plsc

MAX_LEN = 4096
D_MODEL = 1024
BATCH = 4

NUM_CORES = 2
NUM_SUBCORES = 16
NUM_WORKERS = NUM_CORES * NUM_SUBCORES
ROWS_PER_WORKER = MAX_LEN // NUM_WORKERS
CHUNK = 32
NUM_CHUNKS = ROWS_PER_WORKER // CHUNK


def _sc_broadcast(table_hbm, out_hbm, buf0, buf1, buf2, gsem, ssem0, ssem1, ssem2):
    wid = lax.axis_index("s") * NUM_CORES + lax.axis_index("c")
    base = wid * ROWS_PER_WORKER
    bufs = (buf0, buf1, buf2)
    ssems = (ssem0, ssem1, ssem2)

    def gather(c):
        row = base + c * CHUNK
        return pltpu.async_copy(table_hbm.at[pl.ds(row, CHUNK)], bufs[c % 3], gsem)

    gathers = {0: gather(0)}
    scatters = {}
    for c in range(NUM_CHUNKS):
        row = base + c * CHUNK
        gathers[c].wait()
        scatters[c] = [
            pltpu.async_copy(bufs[c % 3], out_hbm.at[b, pl.ds(row, CHUNK)], ssems[c % 3])
            for b in range(BATCH)
        ]
        if c + 1 < NUM_CHUNKS:
            if c - 2 >= 0:
                for cp in scatters[c - 2]:
                    cp.wait()
            gathers[c + 1] = gather(c + 1)
    for c in range(max(0, NUM_CHUNKS - 2), NUM_CHUNKS):
        for cp in scatters[c]:
            cp.wait()


def _sc_broadcast_direct(table_hbm, out_hbm, sem):
    wid = lax.axis_index("s") * NUM_CORES + lax.axis_index("c")
    base = wid * ROWS_PER_WORKER
    sl = pl.ds(base, ROWS_PER_WORKER)
    copies = [
        pltpu.async_copy(table_hbm.at[sl], out_hbm.at[b, sl], sem)
        for b in range(BATCH)
    ]
    for cp in copies:
        cp.wait()


TC_CHUNK = 512
TC_NCHUNK = MAX_LEN // TC_CHUNK


def _tc_dma_body(table_hbm, out_hbm, buf, gsem, ssem):
    for c in range(TC_NCHUNK):
        row = c * TC_CHUNK
        g = pltpu.make_async_copy(table_hbm.at[pl.ds(row, TC_CHUNK)], buf, gsem)
        g.start()
        g.wait()
        scatters = []
        for b in range(BATCH):
            cp = pltpu.make_async_copy(
                buf, out_hbm.at[pl.ds(b * MAX_LEN + row, TC_CHUNK)], ssem)
            cp.start()
            scatters.append(cp)
        for cp in scatters:
            cp.wait()


def kernel(x, emb_weight):
    del x
    flat = pl.pallas_call(
        _tc_dma_body,
        in_specs=[pl.BlockSpec(memory_space=pl.ANY)],
        out_specs=pl.BlockSpec(memory_space=pl.ANY),
        out_shape=jax.ShapeDtypeStruct((BATCH * MAX_LEN, D_MODEL), jnp.float32),
        scratch_shapes=[
            pltpu.VMEM((TC_CHUNK, D_MODEL), jnp.float32),
            pltpu.SemaphoreType.DMA,
            pltpu.SemaphoreType.DMA,
        ],
    )(emb_weight)
    return flat.reshape(BATCH, MAX_LEN, D_MODEL)
